# baseline (speedup 1.0000x reference)
import jax
import jax.numpy as jnp
from jax import lax
from jax.experimental import pallas as pl
from jax.experimental.pallas import tpu as pltpu
from jax.experimental.pallas import tpu_sc as plsc

N_ENV = 256
BUF = 2048
N_OBS = 64
N_ACT = 16
BATCH = 256

NC = 2
NS = 16
L = 16
NW = NC * NS
E_PER = N_ENV // NW
IDX_CHUNK = 128
NCHUNK = BATCH // IDX_CHUNK


def _body(obs_hbm, act_hbm, rew_hbm, don_hbm, trn_hbm, nxt_hbm, idx_hbm,
          obs_o, nxt_o, act_o, rew_o, don_o, trn_o,
          idx_loc, idx_glb,
          obs_b, nxt_b, act_b, rew_b, don_b, trn_b, sem):
    wid = lax.axis_index("c") * NS + lax.axis_index("s")
    for e in range(E_PER):
        env = wid * E_PER + e
        pltpu.sync_copy(idx_hbm.at[env], idx_loc)
        base = env * BUF
        for j in range(BATCH // L):
            c, o = divmod(j, IDX_CHUNK // L)
            v = idx_loc[c, pl.ds(o * L, L)]
            idx_glb[c, pl.ds(o * L, L)] = v + base
        copies = []
        for c in range(NCHUNK):
            s = pl.ds(c * IDX_CHUNK, IDX_CHUNK)
            ig = idx_glb.at[c]
            copies.append(pltpu.async_copy(obs_hbm.at[ig], obs_b.at[s], sem))
            copies.append(pltpu.async_copy(nxt_hbm.at[ig], nxt_b.at[s], sem))
            copies.append(pltpu.async_copy(act_hbm.at[ig], act_b.at[s], sem))
            copies.append(pltpu.async_copy(rew_hbm.at[ig], rew_b.at[s], sem))
            copies.append(pltpu.async_copy(don_hbm.at[ig], don_b.at[s], sem))
            copies.append(pltpu.async_copy(trn_hbm.at[ig], trn_b.at[s], sem))
        for cp in copies:
            cp.wait()
        rowb = env * BATCH
        pltpu.sync_copy(obs_b, obs_o.at[pl.ds(rowb, BATCH)])
        pltpu.sync_copy(nxt_b, nxt_o.at[pl.ds(rowb, BATCH)])
        pltpu.sync_copy(act_b, act_o.at[pl.ds(rowb, BATCH)])
        pltpu.sync_copy(rew_b, rew_o.at[pl.ds(rowb, BATCH)])
        pltpu.sync_copy(don_b, don_o.at[pl.ds(rowb, BATCH)])
        pltpu.sync_copy(trn_b, trn_o.at[pl.ds(rowb, BATCH)])


def kernel(observations, actions, rewards, dones, truncations,
           next_observations, indices):
    n_env, buf, n_obs = observations.shape
    n_act = actions.shape[-1]
    batch = indices.shape[1]
    idt = dones.dtype

    obs_flat = observations.reshape(n_env * buf, n_obs)
    nxt_flat = next_observations.reshape(n_env * buf, n_obs)
    act_flat = actions.reshape(n_env * buf, n_act)
    rew_flat = rewards.reshape(n_env * buf)
    don_flat = dones.reshape(n_env * buf)
    trn_flat = truncations.reshape(n_env * buf)
    idx3 = indices.astype(jnp.int32).reshape(n_env, NCHUNK, IDX_CHUNK)

    mesh = plsc.VectorSubcoreMesh(
        core_axis_name="c", subcore_axis_name="s",
        num_cores=NC, num_subcores=NS)
    f = pl.kernel(
        _body,
        out_type=(
            jax.ShapeDtypeStruct((n_env * batch, n_obs), jnp.float32),
            jax.ShapeDtypeStruct((n_env * batch, n_obs), jnp.float32),
            jax.ShapeDtypeStruct((n_env * batch, n_act), jnp.float32),
            jax.ShapeDtypeStruct((n_env * batch,), jnp.float32),
            jax.ShapeDtypeStruct((n_env * batch,), idt),
            jax.ShapeDtypeStruct((n_env * batch,), idt),
        ),
        mesh=mesh,
        compiler_params=pltpu.CompilerParams(use_tc_tiling_on_sc=False),
        scratch_types=[
            pltpu.VMEM((NCHUNK, IDX_CHUNK), jnp.int32),
            pltpu.VMEM((NCHUNK, IDX_CHUNK), jnp.int32),
            pltpu.VMEM((BATCH, N_OBS), jnp.float32),
            pltpu.VMEM((BATCH, N_OBS), jnp.float32),
            pltpu.VMEM((BATCH, N_ACT), jnp.float32),
            pltpu.VMEM((BATCH,), jnp.float32),
            pltpu.VMEM((BATCH,), idt),
            pltpu.VMEM((BATCH,), idt),
            pltpu.SemaphoreType.DMA,
        ],
    )
    return f(obs_flat, act_flat, rew_flat, don_flat, trn_flat, nxt_flat, idx3)
